# baseline (speedup 1.0000x reference)
import functools

import jax
import jax.numpy as jnp
from jax import lax
from jax.experimental import pallas as pl
from jax.experimental.pallas import tpu as pltpu
from jax.experimental.pallas import tpu_sc as plsc

N = 10000
E = 320000
HID = 16
NC = 2
NS = 16
NW = NC * NS
EPT = E // NW
C = 80
NCH = EPT // C
NPAD = 10240
NPS = NPAD // NS

_F32 = jnp.float32
_DNT = (((1,), (1,)), ((), ()))


def _tc1_body(x_ref, w1_ref, b1_ref, gw_ref, al_ref, ar_ref,
              feat_ref, el_ref, er_ref):
    x16 = lax.dot_general(x_ref[...], w1_ref[...], _DNT,
                          preferred_element_type=_F32)
    x16 = x16 + b1_ref[...]
    feat = lax.dot_general(x16, gw_ref[...], _DNT,
                           preferred_element_type=_F32)
    feat_ref[...] = feat
    el_ref[...] = jnp.sum(feat * al_ref[...], axis=-1, keepdims=True)
    er_ref[...] = jnp.sum(feat * ar_ref[...], axis=-1, keepdims=True)


def _tc2_body(np_ref, sp_ref, gb_ref, wih_ref, bih_ref, bhh_ref,
              gw_ref, al_ref, ar_ref,
              feat_ref, el_ref, er_ref, h_ref, c_ref):
    numer = np_ref[0, :N] + np_ref[1, :N]
    ssum = sp_ref[0, :N] + sp_ref[1, :N]
    gat = numer / (ssum[:, None] + 1e-16) + gb_ref[...]
    xt = jnp.tanh(gat)
    gates = lax.dot_general(xt, wih_ref[...], _DNT,
                            preferred_element_type=_F32)
    gates = gates + bih_ref[...] + bhh_ref[...]
    i_g = jax.nn.sigmoid(gates[:, 0:16])
    g_g = jnp.tanh(gates[:, 32:48])
    o_g = jax.nn.sigmoid(gates[:, 48:64])
    c_new = i_g * g_g
    h_new = o_g * jnp.tanh(c_new)
    c_ref[...] = c_new
    h_ref[...] = h_new
    feat = lax.dot_general(h_new, gw_ref[...], _DNT,
                           preferred_element_type=_F32)
    feat_ref[...] = feat
    el_ref[...] = jnp.sum(feat * al_ref[...], axis=-1, keepdims=True)
    er_ref[...] = jnp.sum(feat * ar_ref[...], axis=-1, keepdims=True)


def _tc3_body(np_ref, sp_ref, gb_ref, wih_ref, whh_ref, bih_ref, bhh_ref,
              h_ref, c_ref, w2_ref, b2_ref, out_ref):
    numer = np_ref[0, :N] + np_ref[1, :N]
    ssum = sp_ref[0, :N] + sp_ref[1, :N]
    gat = numer / (ssum[:, None] + 1e-16) + gb_ref[...]
    xt = jnp.tanh(gat)
    gates = lax.dot_general(xt, wih_ref[...], _DNT,
                            preferred_element_type=_F32)
    gates = gates + bih_ref[...] + bhh_ref[...]
    gates = gates + lax.dot_general(h_ref[...], whh_ref[...], _DNT,
                                    preferred_element_type=_F32)
    i_g = jax.nn.sigmoid(gates[:, 0:16])
    f_g = jax.nn.sigmoid(gates[:, 16:32])
    g_g = jnp.tanh(gates[:, 32:48])
    o_g = jax.nn.sigmoid(gates[:, 48:64])
    c_new = f_g * c_ref[...] + i_g * g_g
    h_new = o_g * jnp.tanh(c_new)
    out = lax.dot_general(h_new, w2_ref[...], _DNT,
                          preferred_element_type=_F32)
    out_ref[...] = out + b2_ref[...]


_tc1 = pl.pallas_call(
    _tc1_body,
    out_shape=[jax.ShapeDtypeStruct((N, HID), _F32),
               jax.ShapeDtypeStruct((N, 1), _F32),
               jax.ShapeDtypeStruct((N, 1), _F32)],
)

_tc2 = pl.pallas_call(
    _tc2_body,
    out_shape=[jax.ShapeDtypeStruct((N, HID), _F32),
               jax.ShapeDtypeStruct((N, 1), _F32),
               jax.ShapeDtypeStruct((N, 1), _F32),
               jax.ShapeDtypeStruct((N, HID), _F32),
               jax.ShapeDtypeStruct((N, HID), _F32)],
)

_tc3 = pl.pallas_call(
    _tc3_body,
    out_shape=[jax.ShapeDtypeStruct((N, 128), _F32)],
)


NBUF = 4


@functools.partial(
    pl.kernel,
    mesh=plsc.VectorSubcoreMesh(core_axis_name="c", subcore_axis_name="s"),
    compiler_params=pltpu.CompilerParams(needs_layout_passes=False,
                                         use_tc_tiling_on_sc=False),
    out_type=[jax.ShapeDtypeStruct((NC, NPAD), _F32),
              jax.ShapeDtypeStruct((NC, NPAD, HID), _F32)],
    scratch_types=[
        pltpu.VMEM((N,), _F32),
        pltpu.VMEM((N,), _F32),
        pltpu.VMEM((EPT,), jnp.int32),
        pltpu.VMEM((EPT,), jnp.int32),
        pltpu.VMEM((EPT,), _F32),
        [pltpu.VMEM((C, HID), _F32)] * NBUF,
        [pltpu.VMEM((C, HID), _F32)] * NBUF,
        pltpu.VMEM_SHARED((NPAD,), _F32),
        pltpu.VMEM_SHARED((NPAD, HID), _F32),
        [pltpu.SemaphoreType.DMA] * NBUF,
        [pltpu.SemaphoreType.DMA] * NBUF,
        pltpu.SemaphoreType.DMA,
    ],
)
def _sc_edge(feat_hbm, el_hbm, er_hbm, src_hbm, dst_hbm,
             z1_hbm, z2_hbm,
             ssum_out, numer_out,
             el_v, er_v, src_f, dst_f, ex_f, gbuf, sbuf,
             ssum_sh, numer_sh, gsem, ssem, esem):
    cid = lax.axis_index("c")
    sid = lax.axis_index("s")
    wid = cid * NS + sid

    a0 = pltpu.async_copy(el_hbm, el_v, gsem[0])
    a1 = pltpu.async_copy(er_hbm, er_v, gsem[1])
    a2 = pltpu.async_copy(src_hbm.at[wid], src_f, gsem[2])
    a3 = pltpu.async_copy(dst_hbm.at[wid], dst_f, gsem[3])
    pltpu.sync_copy(z1_hbm, ssum_sh.at[pl.ds(sid * NPS, NPS)])
    pltpu.sync_copy(z2_hbm, numer_sh.at[pl.ds(sid * NPS, NPS)])
    a0.wait()
    a1.wait()
    a2.wait()
    a3.wait()
    plsc.subcore_barrier()

    def _g_start(j, t):
        pltpu.async_copy(feat_hbm.at[src_f.at[pl.ds(j * C, C)]],
                         gbuf[t], gsem[t])

    def _g_wait(j, t):
        pltpu.make_async_copy(feat_hbm.at[src_f.at[pl.ds(j * C, C)]],
                              gbuf[t], gsem[t]).wait()

    def _s_start(j, t):
        pltpu.async_copy(sbuf[t], numer_sh.at[dst_f.at[pl.ds(j * C, C)]],
                         ssem[t], add=True)

    def _s_wait(j, t):
        pltpu.make_async_copy(sbuf[t], numer_sh.at[dst_f.at[pl.ds(j * C, C)]],
                              ssem[t]).wait()

    for t in range(NBUF):
        _g_start(t, t)

    def _expA(j, carry):
        for g in range(C // 16):
            s16 = src_f[pl.ds(j * C + g * 16, 16)]
            d16 = dst_f[pl.ds(j * C + g * 16, 16)]
            e = plsc.load_gather(el_v, [s16]) + plsc.load_gather(er_v, [d16])
            e = jnp.where(e > 0.0, e, 0.2 * e)
            ex_f[pl.ds(j * C + g * 16, 16)] = jnp.exp(e)
        return carry

    lax.fori_loop(0, NCH, _expA, 0)

    edesc = pltpu.async_copy(ex_f, ssum_sh.at[dst_f], esem, add=True)

    def _scale(j, t):
        for g in range(C // 16):
            exg = ex_f[pl.ds(j * C + g * 16, 16)]
            for r in range(16):
                rr = g * 16 + r
                sbuf[t][rr, :] = gbuf[t][rr, :] * exg[r]

    def _loop(i, carry):
        for t in range(NBUF):
            j = NBUF * i + t
            _g_wait(j, t)

            @pl.when(i >= 1)
            def _():
                _s_wait(j, t)

            _scale(j, t)
            _s_start(j, t)

            @pl.when(j + NBUF < NCH)
            def _():
                _g_start(j + NBUF, t)
        return carry

    lax.fori_loop(0, NCH // NBUF, _loop, 0)

    jl = NCH - 1
    _g_wait(jl, 0)
    _s_wait(jl, 0)
    _scale(jl, 0)
    _s_start(jl, 0)

    _s_wait(jl, 1)
    _s_wait(jl, 2)
    _s_wait(jl, 3)
    _s_wait(jl, 0)
    edesc.wait()

    plsc.subcore_barrier()
    pltpu.sync_copy(ssum_sh.at[pl.ds(sid * NPS, NPS)],
                    ssum_out.at[cid, pl.ds(sid * NPS, NPS)])
    pltpu.sync_copy(numer_sh.at[pl.ds(sid * NPS, NPS)],
                    numer_out.at[cid, pl.ds(sid * NPS, NPS)])


def kernel(x, edge_index, W1, b1, W2, b2, gat_W, attn_l, attn_r, gat_b,
           W_ih, W_hh, b_ih, b_hh):
    src = edge_index[0].reshape(NW, EPT)
    dst = edge_index[1].reshape(NW, EPT)
    z1 = jnp.zeros((NPS,), _F32)
    z2 = jnp.zeros((NPS, HID), _F32)

    feat0, el0, er0 = _tc1(x, W1, b1[None], gat_W[0],
                           attn_l[0][None], attn_r[0][None])
    ssum0, numer0 = _sc_edge(feat0, el0.reshape(N), er0.reshape(N),
                             src, dst, z1, z2)
    feat1, el1, er1, h0, c0 = _tc2(
        numer0, ssum0, gat_b[0][None],
        W_ih[0], b_ih[0][None], b_hh[0][None],
        gat_W[1], attn_l[1][None], attn_r[1][None])
    ssum1, numer1 = _sc_edge(feat1, el1.reshape(N), er1.reshape(N),
                             src, dst, z1, z2)
    (out,) = _tc3(numer1, ssum1, gat_b[1][None],
                  W_ih[1], W_hh[1], b_ih[1][None], b_hh[1][None],
                  h0, c0, W2, b2[None])
    return out


# speedup vs baseline: 1.0276x; 1.0276x over previous
import functools

import jax
import jax.numpy as jnp
from jax import lax
from jax.experimental import pallas as pl
from jax.experimental.pallas import tpu as pltpu
from jax.experimental.pallas import tpu_sc as plsc

N = 10000
E = 320000
HID = 16
NC = 2
NS = 16
NW = NC * NS
EPT = E // NW
C = 80
NCH = EPT // C
NPAD = 10240
NPS = NPAD // NS

_F32 = jnp.float32
_DNT = (((1,), (1,)), ((), ()))


def _tc1_body(x_ref, w1_ref, b1_ref, gw_ref, al_ref, ar_ref,
              feat_ref, el_ref, er_ref):
    x16 = lax.dot_general(x_ref[...], w1_ref[...], _DNT,
                          preferred_element_type=_F32)
    x16 = x16 + b1_ref[...]
    feat = lax.dot_general(x16, gw_ref[...], _DNT,
                           preferred_element_type=_F32)
    feat_ref[...] = feat
    el_ref[...] = jnp.sum(feat * al_ref[...], axis=-1)
    er_ref[...] = jnp.sum(feat * ar_ref[...], axis=-1)


def _tc2_body(np_ref, sp_ref, gb_ref, wih_ref, bih_ref, bhh_ref,
              gw_ref, al_ref, ar_ref,
              feat_ref, el_ref, er_ref, h_ref, c_ref):
    numer = np_ref[0, :N] + np_ref[1, :N]
    ssum = sp_ref[0, :N] + sp_ref[1, :N]
    gat = numer / (ssum[:, None] + 1e-16) + gb_ref[...]
    xt = jnp.tanh(gat)
    gates = lax.dot_general(xt, wih_ref[...], _DNT,
                            preferred_element_type=_F32)
    gates = gates + bih_ref[...] + bhh_ref[...]
    i_g = jax.nn.sigmoid(gates[:, 0:16])
    g_g = jnp.tanh(gates[:, 32:48])
    o_g = jax.nn.sigmoid(gates[:, 48:64])
    c_new = i_g * g_g
    h_new = o_g * jnp.tanh(c_new)
    c_ref[...] = c_new
    h_ref[...] = h_new
    feat = lax.dot_general(h_new, gw_ref[...], _DNT,
                           preferred_element_type=_F32)
    feat_ref[...] = feat
    el_ref[...] = jnp.sum(feat * al_ref[...], axis=-1)
    er_ref[...] = jnp.sum(feat * ar_ref[...], axis=-1)


def _tc3_body(np_ref, sp_ref, gb_ref, wih_ref, whh_ref, bih_ref, bhh_ref,
              h_ref, c_ref, w2_ref, b2_ref, out_ref):
    numer = np_ref[0, :N] + np_ref[1, :N]
    ssum = sp_ref[0, :N] + sp_ref[1, :N]
    gat = numer / (ssum[:, None] + 1e-16) + gb_ref[...]
    xt = jnp.tanh(gat)
    gates = lax.dot_general(xt, wih_ref[...], _DNT,
                            preferred_element_type=_F32)
    gates = gates + bih_ref[...] + bhh_ref[...]
    gates = gates + lax.dot_general(h_ref[...], whh_ref[...], _DNT,
                                    preferred_element_type=_F32)
    i_g = jax.nn.sigmoid(gates[:, 0:16])
    f_g = jax.nn.sigmoid(gates[:, 16:32])
    g_g = jnp.tanh(gates[:, 32:48])
    o_g = jax.nn.sigmoid(gates[:, 48:64])
    c_new = f_g * c_ref[...] + i_g * g_g
    h_new = o_g * jnp.tanh(c_new)
    out = lax.dot_general(h_new, w2_ref[...], _DNT,
                          preferred_element_type=_F32)
    out_ref[...] = out + b2_ref[...]


_tc1 = pl.pallas_call(
    _tc1_body,
    out_shape=[jax.ShapeDtypeStruct((N, HID), _F32),
               jax.ShapeDtypeStruct((N,), _F32),
               jax.ShapeDtypeStruct((N,), _F32)],
)

_tc2 = pl.pallas_call(
    _tc2_body,
    out_shape=[jax.ShapeDtypeStruct((N, HID), _F32),
               jax.ShapeDtypeStruct((N,), _F32),
               jax.ShapeDtypeStruct((N,), _F32),
               jax.ShapeDtypeStruct((N, HID), _F32),
               jax.ShapeDtypeStruct((N, HID), _F32)],
)

_tc3 = pl.pallas_call(
    _tc3_body,
    out_shape=[jax.ShapeDtypeStruct((N, 128), _F32)],
)


NBUF = 4


@functools.partial(
    pl.kernel,
    mesh=plsc.VectorSubcoreMesh(core_axis_name="c", subcore_axis_name="s"),
    compiler_params=pltpu.CompilerParams(needs_layout_passes=False,
                                         use_tc_tiling_on_sc=False),
    out_type=[jax.ShapeDtypeStruct((NC, NPAD), _F32),
              jax.ShapeDtypeStruct((NC, NPAD, HID), _F32)],
    scratch_types=[
        pltpu.VMEM((N,), _F32),
        pltpu.VMEM((N,), _F32),
        pltpu.VMEM((EPT,), jnp.int32),
        pltpu.VMEM((EPT,), jnp.int32),
        pltpu.VMEM((EPT,), _F32),
        [pltpu.VMEM((C, HID), _F32)] * NBUF,
        [pltpu.VMEM((C, HID), _F32)] * NBUF,
        pltpu.VMEM_SHARED((NPAD,), _F32),
        pltpu.VMEM_SHARED((NPAD, HID), _F32),
        [pltpu.SemaphoreType.DMA] * NBUF,
        [pltpu.SemaphoreType.DMA] * NBUF,
        pltpu.SemaphoreType.DMA,
    ],
)
def _sc_edge(feat_hbm, el_hbm, er_hbm, src_hbm, dst_hbm,
             z1_hbm, z2_hbm,
             ssum_out, numer_out,
             el_v, er_v, src_f, dst_f, ex_f, gbuf, sbuf,
             ssum_sh, numer_sh, gsem, ssem, esem):
    cid = lax.axis_index("c")
    sid = lax.axis_index("s")
    wid = cid * NS + sid

    a0 = pltpu.async_copy(el_hbm, el_v, gsem[0])
    a1 = pltpu.async_copy(er_hbm, er_v, gsem[1])
    a2 = pltpu.async_copy(src_hbm.at[wid], src_f, gsem[2])
    a3 = pltpu.async_copy(dst_hbm.at[wid], dst_f, gsem[3])
    pltpu.sync_copy(z1_hbm, ssum_sh.at[pl.ds(sid * NPS, NPS)])
    pltpu.sync_copy(z2_hbm, numer_sh.at[pl.ds(sid * NPS, NPS)])
    a0.wait()
    a1.wait()
    a2.wait()
    a3.wait()
    plsc.subcore_barrier()

    def _g_start(j, t):
        pltpu.async_copy(feat_hbm.at[src_f.at[pl.ds(j * C, C)]],
                         gbuf[t], gsem[t])

    def _g_wait(j, t):
        pltpu.make_async_copy(feat_hbm.at[src_f.at[pl.ds(j * C, C)]],
                              gbuf[t], gsem[t]).wait()

    def _s_start(j, t):
        pltpu.async_copy(sbuf[t], numer_sh.at[dst_f.at[pl.ds(j * C, C)]],
                         ssem[t], add=True)

    def _s_wait(j, t):
        pltpu.make_async_copy(sbuf[t], numer_sh.at[dst_f.at[pl.ds(j * C, C)]],
                              ssem[t]).wait()

    for t in range(NBUF):
        _g_start(t, t)

    def _expA(j, carry):
        for g in range(C // 16):
            s16 = src_f[pl.ds(j * C + g * 16, 16)]
            d16 = dst_f[pl.ds(j * C + g * 16, 16)]
            e = plsc.load_gather(el_v, [s16]) + plsc.load_gather(er_v, [d16])
            e = jnp.where(e > 0.0, e, 0.2 * e)
            ex_f[pl.ds(j * C + g * 16, 16)] = jnp.exp(e)
        return carry

    lax.fori_loop(0, NCH, _expA, 0)

    edesc = pltpu.async_copy(ex_f, ssum_sh.at[dst_f], esem, add=True)

    def _scale(j, t):
        for g in range(C // 16):
            exg = ex_f[pl.ds(j * C + g * 16, 16)]
            for r in range(16):
                rr = g * 16 + r
                sbuf[t][rr, :] = gbuf[t][rr, :] * exg[r]

    def _loop(i, carry):
        for t in range(NBUF):
            j = NBUF * i + t
            _g_wait(j, t)

            @pl.when(i >= 1)
            def _():
                _s_wait(j, t)

            _scale(j, t)
            _s_start(j, t)

            @pl.when(j + NBUF < NCH)
            def _():
                _g_start(j + NBUF, t)
        return carry

    lax.fori_loop(0, NCH // NBUF, _loop, 0)

    jl = NCH - 1
    _g_wait(jl, 0)
    _s_wait(jl, 0)
    _scale(jl, 0)
    _s_start(jl, 0)

    _s_wait(jl, 1)
    _s_wait(jl, 2)
    _s_wait(jl, 3)
    _s_wait(jl, 0)
    edesc.wait()

    plsc.subcore_barrier()
    pltpu.sync_copy(ssum_sh.at[pl.ds(sid * NPS, NPS)],
                    ssum_out.at[cid, pl.ds(sid * NPS, NPS)])
    pltpu.sync_copy(numer_sh.at[pl.ds(sid * NPS, NPS)],
                    numer_out.at[cid, pl.ds(sid * NPS, NPS)])


def kernel(x, edge_index, W1, b1, W2, b2, gat_W, attn_l, attn_r, gat_b,
           W_ih, W_hh, b_ih, b_hh):
    src = edge_index[0].reshape(NW, EPT)
    dst = edge_index[1].reshape(NW, EPT)
    z1 = jnp.zeros((NPS,), _F32)
    z2 = jnp.zeros((NPS, HID), _F32)

    feat0, el0, er0 = _tc1(x, W1, b1[None], gat_W[0],
                           attn_l[0][None], attn_r[0][None])
    ssum0, numer0 = _sc_edge(feat0, el0, er0, src, dst, z1, z2)
    feat1, el1, er1, h0, c0 = _tc2(
        numer0, ssum0, gat_b[0][None],
        W_ih[0], b_ih[0][None], b_hh[0][None],
        gat_W[1], attn_l[1][None], attn_r[1][None])
    ssum1, numer1 = _sc_edge(feat1, el1, er1, src, dst, z1, z2)
    (out,) = _tc3(numer1, ssum1, gat_b[1][None],
                  W_ih[1], W_hh[1], b_ih[1][None], b_hh[1][None],
                  h0, c0, W2, b2[None])
    return out
